# baseline (speedup 1.0000x reference)
import functools

import jax
import jax.numpy as jnp
from jax import lax
from jax.experimental import pallas as pl
from jax.experimental.pallas import tpu as pltpu
from jax.experimental.pallas import tpu_sc as plsc

N = 10000
E = 320000
F = 128

NPAD = 10240
CH = 128
NCH = 80
NTILES = 32
EPT = NCH * CH
EPAD_TOT = NTILES * EPT
ROWS_PER_TILE = NPAD // 16

EROWS = E // F
PROWS = EPAD_TOT // F

NB = 10
BR = N // NB

_mesh = plsc.VectorSubcoreMesh(core_axis_name="c", subcore_axis_name="s")


EPR = E // NTILES
FULLROWS = EPR // CH
REM = EPR - FULLROWS * CH


def _deg_pack_kernel(ei):
    @functools.partial(
        pl.kernel,
        out_type=[
            jax.ShapeDtypeStruct((2, NPAD), jnp.float32),
            jax.ShapeDtypeStruct((NTILES, NCH, CH), jnp.int32),
        ],
        mesh=_mesh,
        scratch_types=[
            pltpu.VMEM((EPR + CH - REM,), jnp.int32),
            pltpu.VMEM((EPR + CH - REM,), jnp.int32),
            pltpu.VMEM((NCH, CH), jnp.int32),
            pltpu.VMEM((NCH, CH), jnp.int32),
            pltpu.VMEM((CH,), jnp.float32),
            pltpu.VMEM((ROWS_PER_TILE,), jnp.float32),
            pltpu.VMEM_SHARED((NPAD,), jnp.float32),
            pltpu.SemaphoreType.DMA,
            pltpu.SemaphoreType.DMA,
        ],
    )
    def k(ei_hbm, deg_hbm, pk_hbm, src1, dst1, pk_v, dst2, ones_v, zeros_v,
          deg_sh, sem0, sem1):
        c = lax.axis_index("c")
        s = lax.axis_index("s")
        w = c * 16 + s
        off = (w % 8) * REM
        a0 = pl.multiple_of(w * EPR - off, CH)
        nfetch = EPR + CH - REM
        pltpu.async_copy(ei_hbm.at[0].at[pl.ds(a0, nfetch)], src1, sem0)
        pltpu.async_copy(ei_hbm.at[1].at[pl.ds(a0, nfetch)], dst1, sem1)

        @pl.loop(0, CH, step=16)
        def _(i):
            ones_v.at[pl.ds(i, 16)][...] = jnp.full((16,), 1.0, jnp.float32)

        @pl.loop(0, ROWS_PER_TILE, step=16)
        def _(i):
            zeros_v.at[pl.ds(i, 16)][...] = jnp.full((16,), 0.0, jnp.float32)

        base = s * ROWS_PER_TILE
        pltpu.sync_copy(zeros_v, deg_sh.at[pl.ds(base, ROWS_PER_TILE)])
        pltpu.make_async_copy(ei_hbm.at[0].at[pl.ds(a0, nfetch)], src1,
                              sem0).wait()
        pltpu.make_async_copy(ei_hbm.at[1].at[pl.ds(a0, nfetch)], dst1,
                              sem1).wait()

        @pl.loop(0, FULLROWS)
        def _(r):
            @pl.loop(0, CH, step=16)
            def _(col):
                sv = src1.at[pl.ds(off + r * CH + col, 16)][...]
                dv = dst1.at[pl.ds(off + r * CH + col, 16)][...]
                pk_v.at[r, pl.ds(col, 16)][...] = (dv << 16) | sv
                dst2.at[r, pl.ds(col, 16)][...] = dv

        lane = lax.iota(jnp.int32, 16)

        @pl.loop(FULLROWS, NCH)
        def _(r):
            @pl.loop(0, CH, step=16)
            def _(col):
                flat = r * CH + col
                pdst = N + (flat + lane) % (NPAD - N)
                psrc = ((flat + lane) * 61) % N
                pk = (pdst << 16) | psrc

                @pl.when(jnp.logical_and(r == FULLROWS, col < REM))
                def _():
                    sv = src1.at[pl.ds(off + r * CH + col, 16)][...]
                    dv = dst1.at[pl.ds(off + r * CH + col, 16)][...]
                    pk_v.at[r, pl.ds(col, 16)][...] = (dv << 16) | sv
                    dst2.at[r, pl.ds(col, 16)][...] = dv

                @pl.when(jnp.logical_or(r != FULLROWS, col >= REM))
                def _():
                    pk_v.at[r, pl.ds(col, 16)][...] = pk
                    dst2.at[r, pl.ds(col, 16)][...] = pdst

        pltpu.async_copy(pk_v, pk_hbm.at[w], sem0)
        plsc.subcore_barrier()

        @pl.loop(0, NCH)
        def _(j):
            pltpu.async_copy(ones_v, deg_sh.at[dst2.at[j]], sem1, add=True)

        @pl.loop(0, NCH)
        def _(j):
            pltpu.make_async_copy(ones_v, deg_sh.at[dst2.at[j]], sem1).wait()

        plsc.subcore_barrier()
        pltpu.sync_copy(deg_sh.at[pl.ds(base, ROWS_PER_TILE)],
                        deg_hbm.at[c].at[pl.ds(base, ROWS_PER_TILE)])
        pltpu.make_async_copy(pk_v, pk_hbm.at[w], sem0).wait()

    return k(ei)


def _scatter_kernel(hs, pk_t):
    @functools.partial(
        pl.kernel,
        out_type=jax.ShapeDtypeStruct((2, NPAD, F), jnp.float32),
        mesh=_mesh,
        scratch_types=[
            pltpu.VMEM((NCH, CH), jnp.int32),
            pltpu.VMEM((2, CH), jnp.int32),
            pltpu.VMEM((2, CH), jnp.int32),
            pltpu.VMEM((CH, F), jnp.float32),
            pltpu.VMEM((CH, F), jnp.float32),
            pltpu.VMEM_SHARED((NPAD, F), jnp.float32),
            pltpu.SemaphoreType.DMA,
            pltpu.SemaphoreType.DMA,
        ],
    )
    def k(hs_hbm, pk_hbm, out_hbm, pk_v, si_v, di_v, msg0, msg1,
          acc_sh, sem0, sem1):
        c = lax.axis_index("c")
        s = lax.axis_index("s")
        w = c * 16 + s
        pltpu.async_copy(pk_hbm.at[w], pk_v, sem0)

        def unpack_src(j, slot):
            @pl.loop(0, CH, step=16)
            def _(i):
                p = pk_v.at[j, pl.ds(i, 16)][...]
                si_v.at[slot, pl.ds(i, 16)][...] = jnp.bitwise_and(
                    p, jnp.int32(0xFFFF))

        def unpack_dst(j, slot):
            @pl.loop(0, CH, step=16)
            def _(i):
                p = pk_v.at[j, pl.ds(i, 16)][...]
                di_v.at[slot, pl.ds(i, 16)][...] = lax.shift_right_logical(
                    p, jnp.int32(16))

        @pl.loop(0, CH)
        def _(r):
            @pl.loop(0, F, step=16)
            def _(col):
                msg0.at[r, pl.ds(col, 16)][...] = jnp.full((16,), 0.0,
                                                           jnp.float32)

        base = s * ROWS_PER_TILE
        pltpu.make_async_copy(pk_hbm.at[w], pk_v, sem0).wait()

        @pl.loop(0, ROWS_PER_TILE, step=CH)
        def _(r):
            pltpu.async_copy(msg0, acc_sh.at[pl.ds(base + r, CH)], sem1)

        @pl.loop(0, ROWS_PER_TILE, step=CH)
        def _(r):
            pltpu.make_async_copy(msg0, acc_sh.at[pl.ds(base + r, CH)],
                                  sem1).wait()

        plsc.subcore_barrier()

        unpack_src(0, 0)
        pltpu.async_copy(hs_hbm.at[si_v.at[0]], msg0, sem0)
        unpack_src(1, 1)
        pltpu.async_copy(hs_hbm.at[si_v.at[1]], msg1, sem1)

        @pl.loop(0, NCH, step=2)
        def _(j):
            pltpu.make_async_copy(hs_hbm.at[si_v.at[0]], msg0, sem0).wait()
            unpack_dst(j, 0)
            pltpu.sync_copy(msg0, acc_sh.at[di_v.at[0]], add=True)

            @pl.when(j + 2 < NCH)
            def _():
                unpack_src(j + 2, 0)
                pltpu.async_copy(hs_hbm.at[si_v.at[0]], msg0, sem0)

            pltpu.make_async_copy(hs_hbm.at[si_v.at[1]], msg1, sem1).wait()
            unpack_dst(j + 1, 1)
            pltpu.sync_copy(msg1, acc_sh.at[di_v.at[1]], add=True)

            @pl.when(j + 3 < NCH)
            def _():
                unpack_src(j + 3, 1)
                pltpu.async_copy(hs_hbm.at[si_v.at[1]], msg1, sem1)

        plsc.subcore_barrier()

        @pl.loop(0, ROWS_PER_TILE, step=CH)
        def _(r):
            pltpu.async_copy(acc_sh.at[pl.ds(base + r, CH)],
                             out_hbm.at[c].at[pl.ds(base + r, CH)], sem0)

        @pl.loop(0, ROWS_PER_TILE, step=CH)
        def _(r):
            pltpu.make_async_copy(acc_sh.at[pl.ds(base + r, CH)],
                                  out_hbm.at[c].at[pl.ds(base + r, CH)],
                                  sem0).wait()

    return k(hs, pk_t)


def _sigma_from(W, u):
    u = u / (jnp.sqrt(jnp.sum(u * u)) + 1e-12)
    v = lax.dot_general(u, W, (((1,), (0,)), ((), ())),
                        preferred_element_type=jnp.float32)
    v = v / (jnp.sqrt(jnp.sum(v * v)) + 1e-12)
    Wv = lax.dot_general(v, W, (((1,), (1,)), ((), ())),
                         preferred_element_type=jnp.float32)
    nWv = jnp.sqrt(jnp.sum(Wv * Wv))
    return (nWv * nWv) / (nWv + 1e-12)


def _mm_body(x_ref, W_ref, u_ref, h_ref):
    sigma = _sigma_from(W_ref[...], u_ref[...])
    h_ref[...] = jnp.dot(x_ref[...], W_ref[...] / sigma,
                         preferred_element_type=jnp.float32)


def _mm(x, W, u_r):
    return pl.pallas_call(
        _mm_body,
        out_shape=jax.ShapeDtypeStruct((N, F), jnp.float32),
    )(x, W, u_r)


def _prep_body(degp_ref, dinv_ref):
    deg = degp_ref[0] + degp_ref[1] + 1.0
    dinv_ref[...] = lax.rsqrt(deg)


def _prep(deg_parts):
    return pl.pallas_call(
        _prep_body,
        out_shape=jax.ShapeDtypeStruct((NPAD // F, F), jnp.float32),
    )(deg_parts.reshape(2, NPAD // F, F))


def _scale_body(h_ref, dinv_ref, hs_ref):
    hs_ref[...] = h_ref[...] * dinv_ref[...]


def _scale(h, dinv_col):
    return pl.pallas_call(
        _scale_body,
        out_shape=jax.ShapeDtypeStruct((N, F), jnp.float32),
    )(h, dinv_col)


def _combineA_body(p_ref, hs1_ref, dinv_ref, b_ref, Wx_ref, ux_ref,
                   hs2_ref):
    dinv = dinv_ref[...]
    agg = p_ref[0, pl.ds(0, N), :] + p_ref[1, pl.ds(0, N), :]
    z1 = jnp.maximum(dinv * agg + dinv * hs1_ref[...] + b_ref[...], 0.0)
    sigma = _sigma_from(Wx_ref[...], ux_ref[...])
    h2 = jnp.dot(z1, Wx_ref[...] / sigma,
                 preferred_element_type=jnp.float32)
    hs2_ref[...] = h2 * dinv


def _combineA(p1, hs1, dinv_col, b1r, Wx, uxr):
    return pl.pallas_call(
        _combineA_body,
        out_shape=jax.ShapeDtypeStruct((N, F), jnp.float32),
    )(p1, hs1, dinv_col, b1r, Wx, uxr)


def _combineB_body(p_ref, hs1_ref, dinv_ref, b_ref, z1_ref):
    dinv = dinv_ref[...]
    agg = p_ref[0, pl.ds(0, N), :] + p_ref[1, pl.ds(0, N), :]
    z1_ref[...] = jnp.maximum(dinv * agg + dinv * hs1_ref[...] + b_ref[...],
                              0.0)


def _combineB(p1, hs1, dinv_col, b1r):
    return pl.pallas_call(
        _combineB_body,
        out_shape=jax.ShapeDtypeStruct((N, F), jnp.float32),
    )(p1, hs1, dinv_col, b1r)


def _combine2_body(p_ref, hs2_ref, z1_ref, dinv_ref, b_ref, out_ref):
    dinv = dinv_ref[...]
    agg = p_ref[0, pl.ds(0, N), :] + p_ref[1, pl.ds(0, N), :]
    z2 = jnp.maximum(dinv * agg + dinv * hs2_ref[...] + b_ref[...], 0.0)
    out_ref[...] = jnp.maximum(z1_ref[...], z2)


def _combine2(p2, hs2, z1, dinv_col, bxr):
    return pl.pallas_call(
        _combine2_body,
        out_shape=jax.ShapeDtypeStruct((N, F), jnp.float32),
    )(p2, hs2, z1, dinv_col, bxr)


def kernel(x, edge_index, W1, b1, u1, Wx, bx, ux):
    u1r = u1.reshape(1, F)
    uxr = ux.reshape(1, F)
    b1r = b1.reshape(1, F)
    bxr = bx.reshape(1, F)

    deg_parts, pk_t = _deg_pack_kernel(edge_index)
    h1 = _mm(x, W1, u1r)
    dinv_grid = _prep(deg_parts)
    dinv_col = dinv_grid.reshape(NPAD, 1)[:N]
    hs1 = _scale(h1, dinv_col)

    p1 = _scatter_kernel(hs1, pk_t)
    hs2 = _combineA(p1, hs1, dinv_col, b1r, Wx, uxr)
    p2 = _scatter_kernel(hs2, pk_t)
    z1 = _combineB(p1, hs1, dinv_col, b1r)
    out = _combine2(p2, hs2, z1, dinv_col, bxr)
    return out


# speedup vs baseline: 1.0061x; 1.0061x over previous
import functools

import jax
import jax.numpy as jnp
from jax import lax
from jax.experimental import pallas as pl
from jax.experimental.pallas import tpu as pltpu
from jax.experimental.pallas import tpu_sc as plsc

N = 10000
E = 320000
F = 128

NPAD = 10240
CH = 128
NCH = 80
NTILES = 32
EPT = NCH * CH
EPAD_TOT = NTILES * EPT
ROWS_PER_TILE = NPAD // 16

EROWS = E // F
PROWS = EPAD_TOT // F

NB = 10
BR = N // NB

_mesh = plsc.VectorSubcoreMesh(core_axis_name="c", subcore_axis_name="s")


EPR = E // NTILES
FULLROWS = EPR // CH
REM = EPR - FULLROWS * CH


def _deg_pack_kernel(ei):
    @functools.partial(
        pl.kernel,
        out_type=[
            jax.ShapeDtypeStruct((2, NPAD), jnp.float32),
            jax.ShapeDtypeStruct((NTILES, NCH, CH), jnp.int32),
        ],
        mesh=_mesh,
        scratch_types=[
            pltpu.VMEM((EPR + CH - REM,), jnp.int32),
            pltpu.VMEM((EPR + CH - REM,), jnp.int32),
            pltpu.VMEM((NCH, CH), jnp.int32),
            pltpu.VMEM((NCH, CH), jnp.int32),
            pltpu.VMEM((CH,), jnp.float32),
            pltpu.VMEM((ROWS_PER_TILE,), jnp.float32),
            pltpu.VMEM_SHARED((NPAD,), jnp.float32),
            pltpu.SemaphoreType.DMA,
            pltpu.SemaphoreType.DMA,
        ],
    )
    def k(ei_hbm, deg_hbm, pk_hbm, src1, dst1, pk_v, dst2, ones_v, zeros_v,
          deg_sh, sem0, sem1):
        c = lax.axis_index("c")
        s = lax.axis_index("s")
        w = c * 16 + s
        off = (w % 8) * REM
        a0 = pl.multiple_of(w * EPR - off, CH)
        nfetch = EPR + CH - REM
        pltpu.async_copy(ei_hbm.at[0].at[pl.ds(a0, nfetch)], src1, sem0)
        pltpu.async_copy(ei_hbm.at[1].at[pl.ds(a0, nfetch)], dst1, sem1)

        @pl.loop(0, CH, step=16)
        def _(i):
            ones_v.at[pl.ds(i, 16)][...] = jnp.full((16,), 1.0, jnp.float32)

        @pl.loop(0, ROWS_PER_TILE, step=16)
        def _(i):
            zeros_v.at[pl.ds(i, 16)][...] = jnp.full((16,), 0.0, jnp.float32)

        base = s * ROWS_PER_TILE
        pltpu.sync_copy(zeros_v, deg_sh.at[pl.ds(base, ROWS_PER_TILE)])
        pltpu.make_async_copy(ei_hbm.at[0].at[pl.ds(a0, nfetch)], src1,
                              sem0).wait()
        pltpu.make_async_copy(ei_hbm.at[1].at[pl.ds(a0, nfetch)], dst1,
                              sem1).wait()

        @pl.loop(0, FULLROWS)
        def _(r):
            @pl.loop(0, CH, step=16)
            def _(col):
                sv = src1.at[pl.ds(off + r * CH + col, 16)][...]
                dv = dst1.at[pl.ds(off + r * CH + col, 16)][...]
                pk_v.at[r, pl.ds(col, 16)][...] = (dv << 16) | sv
                dst2.at[r, pl.ds(col, 16)][...] = dv

        lane = lax.iota(jnp.int32, 16)

        @pl.loop(FULLROWS, NCH)
        def _(r):
            @pl.loop(0, CH, step=16)
            def _(col):
                flat = r * CH + col
                pdst = N + (flat + lane) % (NPAD - N)
                psrc = ((flat + lane) * 61) % N
                pk = (pdst << 16) | psrc

                @pl.when(jnp.logical_and(r == FULLROWS, col < REM))
                def _():
                    sv = src1.at[pl.ds(off + r * CH + col, 16)][...]
                    dv = dst1.at[pl.ds(off + r * CH + col, 16)][...]
                    pk_v.at[r, pl.ds(col, 16)][...] = (dv << 16) | sv
                    dst2.at[r, pl.ds(col, 16)][...] = dv

                @pl.when(jnp.logical_or(r != FULLROWS, col >= REM))
                def _():
                    pk_v.at[r, pl.ds(col, 16)][...] = pk
                    dst2.at[r, pl.ds(col, 16)][...] = pdst

        pltpu.async_copy(pk_v, pk_hbm.at[w], sem0)
        plsc.subcore_barrier()

        @pl.loop(0, NCH)
        def _(j):
            pltpu.async_copy(ones_v, deg_sh.at[dst2.at[j]], sem1, add=True)

        @pl.loop(0, NCH)
        def _(j):
            pltpu.make_async_copy(ones_v, deg_sh.at[dst2.at[j]], sem1).wait()

        plsc.subcore_barrier()
        pltpu.sync_copy(deg_sh.at[pl.ds(base, ROWS_PER_TILE)],
                        deg_hbm.at[c].at[pl.ds(base, ROWS_PER_TILE)])
        pltpu.make_async_copy(pk_v, pk_hbm.at[w], sem0).wait()

    return k(ei)


def _scatter_kernel(hs, pk_t):
    @functools.partial(
        pl.kernel,
        out_type=jax.ShapeDtypeStruct((2, NPAD, F), jnp.float32),
        mesh=_mesh,
        scratch_types=[
            pltpu.VMEM((NCH, CH), jnp.int32),
            pltpu.VMEM((2, CH), jnp.int32),
            pltpu.VMEM((2, CH), jnp.int32),
            pltpu.VMEM((CH, F), jnp.float32),
            pltpu.VMEM((CH, F), jnp.float32),
            pltpu.VMEM_SHARED((NPAD, F), jnp.float32),
            pltpu.SemaphoreType.DMA,
            pltpu.SemaphoreType.DMA,
        ],
    )
    def k(hs_hbm, pk_hbm, out_hbm, pk_v, si_v, di_v, msg0, msg1,
          acc_sh, sem0, sem1):
        c = lax.axis_index("c")
        s = lax.axis_index("s")
        w = c * 16 + s
        pltpu.sync_copy(pk_hbm.at[w], pk_v)

        def unpack_src(j, slot):
            @pl.loop(0, CH, step=16)
            def _(i):
                p = pk_v.at[j, pl.ds(i, 16)][...]
                si_v.at[slot, pl.ds(i, 16)][...] = jnp.bitwise_and(
                    p, jnp.int32(0xFFFF))

        def unpack_dst(j, slot):
            @pl.loop(0, CH, step=16)
            def _(i):
                p = pk_v.at[j, pl.ds(i, 16)][...]
                di_v.at[slot, pl.ds(i, 16)][...] = lax.shift_right_logical(
                    p, jnp.int32(16))

        @pl.loop(0, CH)
        def _(r):
            @pl.loop(0, F, step=16)
            def _(col):
                msg0.at[r, pl.ds(col, 16)][...] = jnp.full((16,), 0.0,
                                                           jnp.float32)

        base = s * ROWS_PER_TILE

        @pl.loop(0, ROWS_PER_TILE, step=CH)
        def _(r):
            pltpu.sync_copy(msg0, acc_sh.at[pl.ds(base + r, CH)])

        plsc.subcore_barrier()

        unpack_src(0, 0)
        pltpu.async_copy(hs_hbm.at[si_v.at[0]], msg0, sem0)
        unpack_src(1, 1)
        pltpu.async_copy(hs_hbm.at[si_v.at[1]], msg1, sem1)

        @pl.loop(0, NCH, step=2)
        def _(j):
            pltpu.make_async_copy(hs_hbm.at[si_v.at[0]], msg0, sem0).wait()
            unpack_dst(j, 0)
            pltpu.sync_copy(msg0, acc_sh.at[di_v.at[0]], add=True)

            @pl.when(j + 2 < NCH)
            def _():
                unpack_src(j + 2, 0)
                pltpu.async_copy(hs_hbm.at[si_v.at[0]], msg0, sem0)

            pltpu.make_async_copy(hs_hbm.at[si_v.at[1]], msg1, sem1).wait()
            unpack_dst(j + 1, 1)
            pltpu.sync_copy(msg1, acc_sh.at[di_v.at[1]], add=True)

            @pl.when(j + 3 < NCH)
            def _():
                unpack_src(j + 3, 1)
                pltpu.async_copy(hs_hbm.at[si_v.at[1]], msg1, sem1)

        plsc.subcore_barrier()

        @pl.loop(0, ROWS_PER_TILE, step=CH)
        def _(r):
            pltpu.sync_copy(acc_sh.at[pl.ds(base + r, CH)],
                            out_hbm.at[c].at[pl.ds(base + r, CH)])

    return k(hs, pk_t)


def _sigma_from(W, u):
    u = u / (jnp.sqrt(jnp.sum(u * u)) + 1e-12)
    v = lax.dot_general(u, W, (((1,), (0,)), ((), ())),
                        preferred_element_type=jnp.float32)
    v = v / (jnp.sqrt(jnp.sum(v * v)) + 1e-12)
    Wv = lax.dot_general(v, W, (((1,), (1,)), ((), ())),
                         preferred_element_type=jnp.float32)
    nWv = jnp.sqrt(jnp.sum(Wv * Wv))
    return (nWv * nWv) / (nWv + 1e-12)


def _mm_body(x_ref, W_ref, u_ref, h_ref):
    sigma = _sigma_from(W_ref[...], u_ref[...])
    h_ref[...] = jnp.dot(x_ref[...], W_ref[...] / sigma,
                         preferred_element_type=jnp.float32)


def _mm(x, W, u_r):
    return pl.pallas_call(
        _mm_body,
        out_shape=jax.ShapeDtypeStruct((N, F), jnp.float32),
    )(x, W, u_r)


def _prep_body(degp_ref, dinv_ref):
    deg = degp_ref[0] + degp_ref[1] + 1.0
    dinv_ref[...] = lax.rsqrt(deg)


def _prep(deg_parts):
    return pl.pallas_call(
        _prep_body,
        out_shape=jax.ShapeDtypeStruct((NPAD // F, F), jnp.float32),
    )(deg_parts.reshape(2, NPAD // F, F))


def _scale_body(h_ref, dinv_ref, hs_ref):
    hs_ref[...] = h_ref[...] * dinv_ref[...]


def _scale(h, dinv_col):
    return pl.pallas_call(
        _scale_body,
        out_shape=jax.ShapeDtypeStruct((N, F), jnp.float32),
    )(h, dinv_col)


def _combine1_body(p_ref, hs1_ref, dinv_ref, b_ref, Wx_ref, ux_ref,
                   z1_ref, hs2_ref):
    dinv = dinv_ref[...]
    agg = p_ref[0, pl.ds(0, N), :] + p_ref[1, pl.ds(0, N), :]
    z1 = jnp.maximum(dinv * agg + dinv * hs1_ref[...] + b_ref[...], 0.0)
    z1_ref[...] = z1
    sigma = _sigma_from(Wx_ref[...], ux_ref[...])
    h2 = jnp.dot(z1, Wx_ref[...] / sigma,
                 preferred_element_type=jnp.float32)
    hs2_ref[...] = h2 * dinv


def _combine1(p1, hs1, dinv_col, b1r, Wx, uxr):
    return pl.pallas_call(
        _combine1_body,
        out_shape=[
            jax.ShapeDtypeStruct((N, F), jnp.float32),
            jax.ShapeDtypeStruct((N, F), jnp.float32),
        ],
    )(p1, hs1, dinv_col, b1r, Wx, uxr)


def _combine2_body(p_ref, hs2_ref, z1_ref, dinv_ref, b_ref, out_ref):
    dinv = dinv_ref[...]
    agg = p_ref[0, pl.ds(0, N), :] + p_ref[1, pl.ds(0, N), :]
    z2 = jnp.maximum(dinv * agg + dinv * hs2_ref[...] + b_ref[...], 0.0)
    out_ref[...] = jnp.maximum(z1_ref[...], z2)


def _combine2(p2, hs2, z1, dinv_col, bxr):
    return pl.pallas_call(
        _combine2_body,
        out_shape=jax.ShapeDtypeStruct((N, F), jnp.float32),
    )(p2, hs2, z1, dinv_col, bxr)


def kernel(x, edge_index, W1, b1, u1, Wx, bx, ux):
    u1r = u1.reshape(1, F)
    uxr = ux.reshape(1, F)
    b1r = b1.reshape(1, F)
    bxr = bx.reshape(1, F)

    deg_parts, pk_t = _deg_pack_kernel(edge_index)
    h1 = _mm(x, W1, u1r)
    dinv_grid = _prep(deg_parts)
    dinv_col = dinv_grid.reshape(NPAD, 1)[:N]
    hs1 = _scale(h1, dinv_col)

    p1 = _scatter_kernel(hs1, pk_t)
    z1, hs2 = _combine1(p1, hs1, dinv_col, b1r, Wx, uxr)
    p2 = _scatter_kernel(hs2, pk_t)
    out = _combine2(p2, hs2, z1, dinv_col, bxr)
    return out
